# baseline (device time: 359147 ns/iter reference)
import functools

import jax
import jax.numpy as jnp
from jax import lax
from jax.experimental import pallas as pl
from jax.experimental.pallas import tpu as pltpu

N_DEV = 16
N_STREAM = 4
N_SLOT = 3
COMM_DTYPE = jnp.bfloat16


def kernel(x, w_mat, scale_x, scale_w):
    m, k_local = x.shape
    _, n = w_mat.shape
    mb = m // N_DEV
    nq = n // N_STREAM

    def body(x_ref, w_ref, sx_ref, sw_ref, out_ref, *scratch):
        comms = scratch[0:4]
        send_sems = scratch[4:8]
        recv_sems = scratch[8:12]
        credits = scratch[12:16]

        my = lax.axis_index("i")
        left = lax.rem(my - 1 + N_DEV, N_DEV)
        right = lax.rem(my + 1, N_DEV)

        fwd = (True, True, False, False)
        dst_dev = tuple(right if f else left for f in fwd)
        up_dev = tuple(left if f else right for f in fwd)

        def contrib(o, k):
            xs = x_ref[pl.ds(o * mb, mb), :]
            ws = w_ref[:, k * nq:(k + 1) * nq]
            return lax.dot_general(
                xs, ws, (((1,), (0,)), ((), ())),
                preferred_element_type=jnp.float32)

        def chunk_at(s, k):
            if fwd[k]:
                return lax.rem(my - 2 - s + 2 * N_DEV, N_DEV)
            return lax.rem(my + 2 + s, N_DEV)

        def make_rdma(s, k):
            return pltpu.make_async_remote_copy(
                src_ref=comms[k].at[(s - 1) % N_SLOT],
                dst_ref=comms[k].at[s % N_SLOT],
                send_sem=send_sems[k].at[(s - 1) % N_SLOT],
                recv_sem=recv_sems[k].at[s % N_SLOT],
                device_id=(dst_dev[k],),
                device_id_type=pl.DeviceIdType.MESH,
            )

        for k in range(N_STREAM):
            comms[k][N_SLOT - 1] = contrib(chunk_at(-1, k), k).astype(COMM_DTYPE)

        barrier_sem = pltpu.get_barrier_semaphore()
        for nbr in (left, right):
            pl.semaphore_signal(barrier_sem, inc=1, device_id=(nbr,),
                                device_id_type=pl.DeviceIdType.MESH)
        pl.semaphore_wait(barrier_sem, 2)

        for k in range(N_STREAM):
            make_rdma(0, k).start()

        scale = sx_ref[0] * sw_ref[0]

        order = (0, 2, 1, 3)
        for s in range(N_DEV - 1):
            for k in order:
                make_rdma(s, k).wait()
                if 1 <= s <= N_DEV - 1 - N_SLOT:
                    pl.semaphore_signal(
                        credits[k], inc=1, device_id=(up_dev[k],),
                        device_id_type=pl.DeviceIdType.MESH)
                merged = (comms[k][s % N_SLOT].astype(jnp.float32)
                          + contrib(chunk_at(s, k), k))
                if s < N_DEV - 2:
                    comms[k][s % N_SLOT] = merged.astype(COMM_DTYPE)
                    if s + 1 >= N_SLOT:
                        pl.semaphore_wait(credits[k], 1)
                    make_rdma(s + 1, k).start()
                else:
                    y = merged * scale
                    out_ref[:, k * nq:(k + 1) * nq] = y * jax.nn.sigmoid(y)

        @functools.partial(pl.run_scoped,
                           second_barrier=pltpu.SemaphoreType.REGULAR)
        def _(second_barrier):
            for nbr in (left, right):
                pl.semaphore_signal(second_barrier, inc=1, device_id=(nbr,),
                                    device_id_type=pl.DeviceIdType.MESH)
            pl.semaphore_wait(second_barrier, 2)

    return pl.pallas_call(
        body,
        out_shape=jax.ShapeDtypeStruct((mb, n), jnp.float32),
        in_specs=[
            pl.BlockSpec(memory_space=pltpu.VMEM),
            pl.BlockSpec(memory_space=pltpu.VMEM),
            pl.BlockSpec(memory_space=pltpu.SMEM),
            pl.BlockSpec(memory_space=pltpu.SMEM),
        ],
        out_specs=pl.BlockSpec(memory_space=pltpu.VMEM),
        scratch_shapes=(
            [pltpu.VMEM((N_SLOT, mb, nq), COMM_DTYPE) for _ in range(N_STREAM)]
            + [pltpu.SemaphoreType.DMA((N_SLOT,)) for _ in range(N_STREAM)]
            + [pltpu.SemaphoreType.DMA((N_SLOT,)) for _ in range(N_STREAM)]
            + [pltpu.SemaphoreType.REGULAR for _ in range(N_STREAM)]
        ),
        compiler_params=pltpu.CompilerParams(collective_id=0),
    )(x, w_mat, scale_x, scale_w)
